# baseline (speedup 1.0000x reference)
import functools

import jax
import jax.numpy as jnp
from jax import lax
from jax.experimental import pallas as pl
from jax.experimental.pallas import tpu as pltpu
from jax.experimental.pallas import tpu_sc as plsc

_D = 32
_NC = 2
_NS = 16
_NW = _NC * _NS
_CH = 128


def _sc_gather4(users3, items3, t_gu, t_gi, t_nu, t_ni, B):
    bpw = B // _NW
    nch = bpw // _CH
    mesh = plsc.VectorSubcoreMesh(core_axis_name="c", subcore_axis_name="s")
    out_t = tuple(
        jax.ShapeDtypeStruct((B, _D), jnp.float32) for _ in range(4)
    )

    @functools.partial(
        pl.kernel,
        out_type=out_t,
        mesh=mesh,
        compiler_params=pltpu.CompilerParams(use_tc_tiling_on_sc=False),
        scratch_types=[
            pltpu.VMEM((nch, _CH), jnp.int32),
            pltpu.VMEM((nch, _CH), jnp.int32),
            pltpu.VMEM((bpw, _D), jnp.float32),
            pltpu.VMEM((bpw, _D), jnp.float32),
            pltpu.VMEM((bpw, _D), jnp.float32),
            pltpu.VMEM((bpw, _D), jnp.float32),
            pltpu.SemaphoreType.DMA,
        ],
    )
    def k(users_h, items_h, tgu, tgi, tnu, tni,
          o_gu, o_gi, o_nu, o_ni,
          idx_u, idx_i, b_gu, b_gi, b_nu, b_ni, sem):
        wid = lax.axis_index("s") * _NC + lax.axis_index("c")
        pltpu.sync_copy(users_h.at[wid], idx_u)
        pltpu.sync_copy(items_h.at[wid], idx_i)
        cps = []
        for j in range(nch):
            dst = pl.ds(j * _CH, _CH)
            cps.append(pltpu.async_copy(tgu.at[idx_u.at[j]], b_gu.at[dst], sem))
            cps.append(pltpu.async_copy(tgi.at[idx_i.at[j]], b_gi.at[dst], sem))
            cps.append(pltpu.async_copy(tnu.at[idx_u.at[j]], b_nu.at[dst], sem))
            cps.append(pltpu.async_copy(tni.at[idx_i.at[j]], b_ni.at[dst], sem))
        for c in cps:
            c.wait()
        row = pl.ds(wid * bpw, bpw)
        pltpu.sync_copy(b_gu, o_gu.at[row])
        pltpu.sync_copy(b_gi, o_gi.at[row])
        pltpu.sync_copy(b_nu, o_nu.at[row])
        pltpu.sync_copy(b_ni, o_ni.at[row])

    return k(users3, items3, t_gu, t_gi, t_nu, t_ni)


def _dense_body(gu, gi, nu, ni, w1u, w1i, b1, wg, wn, out):
    x = (jnp.dot(nu[...], w1u[...], preferred_element_type=jnp.float32)
         + jnp.dot(ni[...], w1i[...], preferred_element_type=jnp.float32)
         + b1[...])
    h = jnp.maximum(x, 0.0)
    g = gu[...] * gi[...]
    out[...] = (jnp.dot(g, wg[...], preferred_element_type=jnp.float32)
                + jnp.dot(h, wn[...], preferred_element_type=jnp.float32))


def kernel(users, items, gmf_user_emb, gmf_item_emb, ncf_user_emb,
           ncf_item_emb, W1, b1, Wout):
    B = users.shape[0]
    bpw = B // _NW
    nch = bpw // _CH
    u3 = users.astype(jnp.int32).reshape(_NW, nch, _CH)
    i3 = items.astype(jnp.int32).reshape(_NW, nch, _CH)
    gu, gi, nu, ni = _sc_gather4(
        u3, i3, gmf_user_emb, gmf_item_emb, ncf_user_emb, ncf_item_emb, B)

    W1T = W1.T
    w1u = W1T[:_D]
    w1i = W1T[_D:]
    b1r = b1.reshape(1, _D)
    wg = Wout[0, :_D].reshape(_D, 1)
    wn = Wout[0, _D:].reshape(_D, 1)

    blk = 2048
    grid = (B // blk,)
    preds = pl.pallas_call(
        _dense_body,
        grid=grid,
        in_specs=[
            pl.BlockSpec((blk, _D), lambda i: (i, 0)),
            pl.BlockSpec((blk, _D), lambda i: (i, 0)),
            pl.BlockSpec((blk, _D), lambda i: (i, 0)),
            pl.BlockSpec((blk, _D), lambda i: (i, 0)),
            pl.BlockSpec((_D, _D), lambda i: (0, 0)),
            pl.BlockSpec((_D, _D), lambda i: (0, 0)),
            pl.BlockSpec((1, _D), lambda i: (0, 0)),
            pl.BlockSpec((_D, 1), lambda i: (0, 0)),
            pl.BlockSpec((_D, 1), lambda i: (0, 0)),
        ],
        out_specs=pl.BlockSpec((blk, 1), lambda i: (i, 0)),
        out_shape=jax.ShapeDtypeStruct((B, 1), jnp.float32),
    )(gu, gi, nu, ni, w1u, w1i, b1r, wg, wn)
    return preds.reshape(B)
